# Initial kernel scaffold; baseline (speedup 1.0000x reference)
import jax
import jax.numpy as jnp
from jax.experimental import pallas as pl


def kernel(x, edge_index, edge_attr, batch, lin_e1_w, lin_e1_b, nn1_w1, nn1_b1, nn1_w2, nn1_b2, lin_e2_w, lin_e2_b, nn2_w1, nn2_b1, nn2_w2, nn2_b2, fc_w, fc_b):
    raise NotImplementedError("write your pallas kernel here")


# speedup vs baseline: 3.3721x; 3.3721x over previous
import functools
import jax
import jax.numpy as jnp
from jax import lax
from jax.experimental import pallas as pl
from jax.experimental.pallas import tpu as pltpu
from jax.experimental.pallas import tpu_sc as plsc

NC = 2
NS = 16
NG = 1000

WE = 512
ZR = 128


def _ceil_to(a, m):
    return (a + m - 1) // m * m


def _zero_zbuf(zbuf):
    def zb(i, _):
        zbuf[i, :] = jnp.zeros((16,), jnp.float32)
        return 0
    lax.fori_loop(0, ZR, zb, 0)


def _edge_window(table, src2d, dst2d, attr2d, acc, sidx, didx, attr_v, rows_v,
                 sem, w0, w1, w2, w3, bv, roff, eoff):
    pltpu.sync_copy(src2d.at[pl.ds(roff, WE // 128)], sidx)
    pltpu.sync_copy(dst2d.at[pl.ds(roff, WE // 128)], didx)
    pltpu.sync_copy(attr2d.at[pl.ds(eoff * 4, WE * 4)], attr_v)
    cps = [pltpu.async_copy(table.at[sidx.at[j]],
                            rows_v.at[pl.ds(j * 128, 128)], sem)
           for j in range(WE // 128)]
    for cp in cps:
        cp.wait()

    def grp(q, _):
        av = attr_v[pl.ds(q * 16, 16)]
        for t in range(4):
            e = q * 4 + t
            r = rows_v[e, :]
            m = jnp.maximum(
                r + av[4 * t] * w0 + av[4 * t + 1] * w1
                + av[4 * t + 2] * w2 + av[4 * t + 3] * w3 + bv, 0.0)
            rows_v[e, :] = m
        return 0
    lax.fori_loop(0, WE // 4, grp, 0)
    for j in range(WE // 128):
        pltpu.sync_copy(rows_v.at[pl.ds(j * 128, 128)],
                        acc.at[didx.at[j]], add=True)


def _zero_acc(acc, zbuf, s, rpt):
    _zero_zbuf(zbuf)
    for t in range(rpt // ZR):
        pltpu.sync_copy(zbuf, acc.at[pl.ds(s * rpt + t * ZR, ZR)])


def _writeback(acc, zbuf, out, s, rpt, obase):
    for t in range(rpt // ZR):
        r0 = s * rpt + t * ZR
        pltpu.sync_copy(acc.at[pl.ds(r0, ZR)], zbuf)
        pltpu.sync_copy(zbuf, out.at[pl.ds(obase + r0, ZR)])


def _sc_scratch(n_pad):
    return [
        pltpu.VMEM_SHARED((n_pad, 16), jnp.float32),
        pltpu.VMEM((ZR, 16), jnp.float32),
        pltpu.VMEM((WE // 128, 128), jnp.int32),
        pltpu.VMEM((WE // 128, 128), jnp.int32),
        pltpu.VMEM((WE * 4,), jnp.float32),
        pltpu.VMEM((WE, 16), jnp.float32),
    ]


def _sc_agg1(n_pad, ew):
    nwin = ew // WE
    rpt = n_pad // NS

    def body(table, src2d, dst2d, attr2d, w_hbm, b_hbm, out,
             acc, zbuf, sidx, didx, attr_v, rows_v, w_v, b_v, sem):
        c = lax.axis_index("c")
        s = lax.axis_index("s")
        pltpu.sync_copy(w_hbm, w_v)
        pltpu.sync_copy(b_hbm, b_v)
        _zero_acc(acc, zbuf, s, rpt)
        plsc.subcore_barrier()

        wid = c * NS + s
        w0 = w_v[0, :]
        w1 = w_v[1, :]
        w2 = w_v[2, :]
        w3 = w_v[3, :]
        bv = b_v[...]

        def win(i, _):
            _edge_window(table, src2d, dst2d, attr2d, acc, sidx, didx,
                         attr_v, rows_v, sem, w0, w1, w2, w3, bv,
                         wid * (ew // 128) + i * (WE // 128),
                         wid * ew + i * WE)
            return 0
        lax.fori_loop(0, nwin, win, 0)
        plsc.subcore_barrier()
        _writeback(acc, zbuf, out, s, rpt, c * n_pad)

    return pl.kernel(
        body,
        out_type=jax.ShapeDtypeStruct((NC * n_pad, 16), jnp.float32),
        mesh=plsc.VectorSubcoreMesh(core_axis_name="c", subcore_axis_name="s"),
        compiler_params=pltpu.CompilerParams(use_tc_tiling_on_sc=False),
        scratch_types=_sc_scratch(n_pad) + [
            pltpu.VMEM((4, 16), jnp.float32),
            pltpu.VMEM((16,), jnp.float32),
            pltpu.SemaphoreType.DMA,
        ],
    )


def _sc_agg2(n_pad, ew2):
    nwin = ew2 // WE
    rpt = n_pad // NS

    def body(tcat, src2d, dst2d, attr2d, w_hbm, b_hbm, out,
             acc, zbuf, sidx, didx, attr_v, rows_v, w_v, b_v, sem):
        c = lax.axis_index("c")
        s = lax.axis_index("s")
        pltpu.sync_copy(w_hbm, w_v)
        pltpu.sync_copy(b_hbm, b_v)

        def slice_body(r, _):
            sl_id = 2 * r + c
            _zero_acc(acc, zbuf, s, rpt)
            plsc.subcore_barrier()

            w0 = w_v[sl_id, 0, :]
            w1 = w_v[sl_id, 1, :]
            w2 = w_v[sl_id, 2, :]
            w3 = w_v[sl_id, 3, :]
            bv = b_v[sl_id, :]
            tbase = sl_id * n_pad

            def win(i, _):
                _edge_window(tcat.at[pl.ds(tbase, n_pad)], src2d, dst2d,
                             attr2d, acc, sidx, didx, attr_v, rows_v, sem,
                             w0, w1, w2, w3, bv,
                             s * (ew2 // 128) + i * (WE // 128),
                             s * ew2 + i * WE)
                return 0
            lax.fori_loop(0, nwin, win, 0)
            plsc.subcore_barrier()
            _writeback(acc, zbuf, out, s, rpt, tbase)
            return 0
        lax.fori_loop(0, 2, slice_body, 0)

    return pl.kernel(
        body,
        out_type=jax.ShapeDtypeStruct((4 * n_pad, 16), jnp.float32),
        mesh=plsc.VectorSubcoreMesh(core_axis_name="c", subcore_axis_name="s"),
        compiler_params=pltpu.CompilerParams(use_tc_tiling_on_sc=False),
        scratch_types=_sc_scratch(n_pad) + [
            pltpu.VMEM((4, 4, 16), jnp.float32),
            pltpu.VMEM((4, 16), jnp.float32),
            pltpu.SemaphoreType.DMA,
        ],
    )


def _sc_pool(n_pad, g_pad):
    npt = n_pad // (NC * NS)
    nwin = npt // 112
    rpt = g_pad // NS

    def body(h2ext, batch2d, out, acc, hbuf, bidx):
        c = lax.axis_index("c")
        s = lax.axis_index("s")

        def zb(i, _):
            for k in range(5):
                hbuf[i, pl.ds(k * 16, 16)] = jnp.zeros((16,), jnp.float32)
            return 0
        lax.fori_loop(0, rpt, zb, 0)
        pltpu.sync_copy(hbuf.at[pl.ds(0, rpt)], acc.at[pl.ds(s * rpt, rpt)])
        plsc.subcore_barrier()

        wid = c * NS + s

        def win(i, _):
            noff = wid * npt + i * 112
            pltpu.sync_copy(h2ext.at[pl.ds(noff, 112)], hbuf)
            pltpu.sync_copy(batch2d.at[pl.ds(wid * nwin + i, 1)], bidx)
            pltpu.sync_copy(hbuf, acc.at[bidx.at[0]], add=True)
            return 0
        lax.fori_loop(0, nwin, win, 0)
        plsc.subcore_barrier()

        pltpu.sync_copy(acc.at[pl.ds(s * rpt, rpt)], hbuf.at[pl.ds(0, rpt)])
        pltpu.sync_copy(hbuf.at[pl.ds(0, rpt)],
                        out.at[pl.ds(c * g_pad + s * rpt, rpt)])

    return pl.kernel(
        body,
        out_type=jax.ShapeDtypeStruct((NC * g_pad, 80), jnp.float32),
        mesh=plsc.VectorSubcoreMesh(core_axis_name="c", subcore_axis_name="s"),
        compiler_params=pltpu.CompilerParams(use_tc_tiling_on_sc=False),
        scratch_types=[
            pltpu.VMEM_SHARED((g_pad, 80), jnp.float32),
            pltpu.VMEM((112, 80), jnp.float32),
            pltpu.VMEM((1, 112), jnp.int32),
        ],
    )


def _mlp1_body(x_ref, a0_ref, a1_ref, w1_ref, b1_ref, w2_ref, b2_ref,
               o0, o1, o2, o3):
    h = x_ref[...] + a0_ref[...] + a1_ref[...]
    h = jnp.maximum(jnp.dot(h, w1_ref[...],
                            preferred_element_type=jnp.float32) + b1_ref[...],
                    0.0)
    h = jnp.maximum(jnp.dot(h, w2_ref[...],
                            preferred_element_type=jnp.float32) + b2_ref[...],
                    0.0)
    o0[...] = h[:, 0:16]
    o1[...] = h[:, 16:32]
    o2[...] = h[:, 32:48]
    o3[...] = h[:, 48:64]


def _mlp2_body(t0, t1, t2, t3, g0, g1, g2, g3, w1_ref, b1_ref, w2_ref, b2_ref,
               o_ref):
    h1 = jnp.concatenate([t0[...], t1[...], t2[...], t3[...]], axis=1)
    agg = jnp.concatenate([g0[...], g1[...], g2[...], g3[...]], axis=1)
    h = h1 + agg
    h = jnp.maximum(jnp.dot(h, w1_ref[...],
                            preferred_element_type=jnp.float32) + b1_ref[...],
                    0.0)
    h = jnp.maximum(jnp.dot(h, w2_ref[...],
                            preferred_element_type=jnp.float32) + b2_ref[...],
                    0.0)
    nrows = h.shape[0]
    ones = jnp.ones((nrows, 1), jnp.float32)
    zeros = jnp.zeros((nrows, 15), jnp.float32)
    o_ref[...] = jnp.concatenate([h, ones, zeros], axis=1)


def _fc_body(p_ref, w_ref, b_ref, o_ref, *, g_pad):
    sums = p_ref[0:g_pad, :] + p_ref[g_pad:2 * g_pad, :]
    counts = jnp.maximum(sums[:, 64:65], 1.0)
    pooled = sums[:, 0:64] / counts
    o_ref[...] = jnp.dot(pooled, w_ref[...],
                         preferred_element_type=jnp.float32) + b_ref[...]


def kernel(x, edge_index, edge_attr, batch, lin_e1_w, lin_e1_b, nn1_w1,
           nn1_b1, nn1_w2, nn1_b2, lin_e2_w, lin_e2_b, nn2_w1, nn2_b1,
           nn2_w2, nn2_b2, fc_w, fc_b):
    n = x.shape[0]
    e = edge_index.shape[1]
    g = NG
    n_pad = _ceil_to(n, 14336)
    e_pad = _ceil_to(e, NC * NS * WE)
    g_pad = _ceil_to(g + 32, 16)
    ew = e_pad // (NC * NS)
    ew2 = e_pad // NS

    f32 = jnp.float32
    i32 = jnp.int32

    src = edge_index[0].astype(i32)
    dst = edge_index[1].astype(i32)
    epad = e_pad - e
    pad_ids = (jnp.arange(epad, dtype=i32) % 64)
    src_p = jnp.concatenate([src, pad_ids]).reshape(e_pad // 128, 128)
    dst_p = jnp.concatenate([dst, n + pad_ids]).reshape(e_pad // 128, 128)
    attr_p = jnp.pad(edge_attr.astype(f32),
                     ((0, epad), (0, 0))).reshape(e_pad * 4)
    x_p = jnp.pad(x.astype(f32), ((0, n_pad - n), (0, 16 - x.shape[1])))
    npad = n_pad - n
    batch_p = jnp.concatenate(
        [batch.astype(i32), g + (jnp.arange(npad, dtype=i32) % 32)]
    ).reshape(n_pad // 112, 112)

    w1e = jnp.pad(lin_e1_w.astype(f32), ((0, 0), (0, 16 - lin_e1_w.shape[1])))
    b1e = jnp.pad(lin_e1_b.astype(f32), (0, 16 - lin_e1_b.shape[0]))
    w2e = lin_e2_w.astype(f32).reshape(4, 4, 16).transpose(1, 0, 2)
    b2e = lin_e2_b.astype(f32).reshape(4, 16)
    nw1 = jnp.pad(nn1_w1.astype(f32), ((0, 16 - nn1_w1.shape[0]), (0, 0)))
    nb1 = nn1_b1.astype(f32).reshape(1, 64)
    nb2 = nn1_b2.astype(f32).reshape(1, 64)
    mb1 = nn2_b1.astype(f32).reshape(1, 64)
    mb2 = nn2_b2.astype(f32).reshape(1, 64)
    fcw = jnp.pad(fc_w.astype(f32), ((0, 0), (0, 16 - fc_w.shape[1])))
    fcb = jnp.pad(fc_b.astype(f32), (0, 16 - fc_b.shape[0])).reshape(1, 16)

    agg1 = _sc_agg1(n_pad, ew)(x_p, src_p, dst_p, attr_p, w1e, b1e)
    a0 = agg1[0:n_pad]
    a1 = agg1[n_pad:2 * n_pad]

    nblk = n_pad // 1024
    row_spec = pl.BlockSpec((1024, 16), lambda i: (i, 0))
    w64_spec = pl.BlockSpec((16, 64), lambda i: (0, 0))
    w6464_spec = pl.BlockSpec((64, 64), lambda i: (0, 0))
    b64_spec = pl.BlockSpec((1, 64), lambda i: (0, 0))
    h1s = pl.pallas_call(
        _mlp1_body,
        grid=(nblk,),
        in_specs=[row_spec, row_spec, row_spec, w64_spec, b64_spec,
                  w6464_spec, b64_spec],
        out_specs=[row_spec] * 4,
        out_shape=[jax.ShapeDtypeStruct((n_pad, 16), f32)] * 4,
    )(x_p, a0, a1, nw1, nb1, nn1_w2.astype(f32), nb2)

    tcat = jnp.concatenate(h1s, axis=0)
    agg2 = _sc_agg2(n_pad, ew2)(tcat, src_p, dst_p, attr_p, w2e, b2e)
    gs = [agg2[j * n_pad:(j + 1) * n_pad] for j in range(4)]

    h2ext = pl.pallas_call(
        _mlp2_body,
        grid=(nblk,),
        in_specs=[row_spec] * 8 + [w6464_spec, b64_spec, w6464_spec, b64_spec],
        out_specs=pl.BlockSpec((1024, 80), lambda i: (i, 0)),
        out_shape=jax.ShapeDtypeStruct((n_pad, 80), f32),
    )(h1s[0], h1s[1], h1s[2], h1s[3], gs[0], gs[1], gs[2], gs[3],
      nn2_w1.astype(f32), mb1, nn2_w2.astype(f32), mb2)

    pooled = _sc_pool(n_pad, g_pad)(h2ext, batch_p)

    out = pl.pallas_call(
        functools.partial(_fc_body, g_pad=g_pad),
        grid=(1,),
        in_specs=[pl.BlockSpec((2 * g_pad, 80), lambda i: (0, 0)),
                  pl.BlockSpec((64, 16), lambda i: (0, 0)),
                  pl.BlockSpec((1, 16), lambda i: (0, 0))],
        out_specs=pl.BlockSpec((g_pad, 16), lambda i: (0, 0)),
        out_shape=jax.ShapeDtypeStruct((g_pad, 16), f32),
    )(pooled, fcw, fcb)

    return out[:g, :fc_w.shape[1]]
